# baseline (speedup 1.0000x reference)
import functools

import jax
import jax.numpy as jnp
from jax import lax
from jax.experimental import pallas as pl
from jax.experimental.pallas import tpu as pltpu
from jax.experimental.pallas import tpu_sc as plsc

N = 10000
E = 320000
D = 128
DE = 4
NP = 10240
BN = 1024
BE = 8000

NTILES = 32
CH = 64
NCH = 159
EPT2 = NCH * CH
E2 = EPT2 * NTILES
BE2 = 5088
NA = 10112
NWB64 = NA // CH


def _prep_body(x_ref, t_ref, hw_ref, hu_ref):
    ids = x_ref[...]
    oh = (ids == lax.broadcasted_iota(jnp.int32, (BN, D), 1)).astype(jnp.float32)
    r = jnp.dot(oh, t_ref[...], preferred_element_type=jnp.float32)
    hw_ref[...] = r[:, :D]
    hu_ref[...] = r[:, D:]


def _edge_body(ea_ref, w_ref, b_ref, ew1_ref, ew2_ref):
    r = jnp.dot(ea_ref[...], w_ref[...], preferred_element_type=jnp.float32)
    r = r + b_ref[...]
    ew1_ref[...] = r[:, :D]
    ew2_ref[...] = r[:, D:]


def _up1_body(hu_ref, agg_ref, wu_ref, b_ref, w2_ref, hw2_ref, hu2_ref):
    agg = agg_ref[0] + agg_ref[1]
    h1 = jnp.maximum(
        hu_ref[...]
        + jnp.dot(agg, wu_ref[...], preferred_element_type=jnp.float32)
        + b_ref[...], 0.0)
    r = jnp.dot(h1, w2_ref[...], preferred_element_type=jnp.float32)
    hw2_ref[...] = r[:, :D]
    hu2_ref[...] = r[:, D:]


def _up2_body(hu2_ref, agg_ref, wu_ref, b_ref, out_ref):
    i = pl.program_id(0)
    agg = agg_ref[0] + agg_ref[1]
    h2 = jnp.maximum(
        hu2_ref[...]
        + jnp.dot(agg, wu_ref[...], preferred_element_type=jnp.float32)
        + b_ref[...], 0.0)
    rid = i * BN + lax.broadcasted_iota(jnp.int32, (BN, 1), 0)
    h2 = jnp.where(rid < N, h2, 0.0)

    @pl.when(i == 0)
    def _():
        out_ref[...] = jnp.zeros_like(out_ref)

    out_ref[...] += jnp.sum(h2, axis=0, keepdims=True)


def _prep(x_pad, t):
    return pl.pallas_call(
        _prep_body,
        grid=(NP // BN,),
        in_specs=[
            pl.BlockSpec((BN, 1), lambda i: (i, 0)),
            pl.BlockSpec((D, 2 * D), lambda i: (0, 0)),
        ],
        out_specs=[
            pl.BlockSpec((BN, D), lambda i: (i, 0)),
            pl.BlockSpec((BN, D), lambda i: (i, 0)),
        ],
        out_shape=[
            jax.ShapeDtypeStruct((NP, D), jnp.float32),
            jax.ShapeDtypeStruct((NP, D), jnp.float32),
        ],
    )(x_pad, t)


def _edge_pre(edge_attr, w, b):
    return pl.pallas_call(
        _edge_body,
        grid=(E2 // BE2,),
        in_specs=[
            pl.BlockSpec((BE2, DE), lambda i: (i, 0)),
            pl.BlockSpec((DE, 2 * D), lambda i: (0, 0)),
            pl.BlockSpec((1, 2 * D), lambda i: (0, 0)),
        ],
        out_specs=[
            pl.BlockSpec((BE2, D), lambda i: (i, 0)),
            pl.BlockSpec((BE2, D), lambda i: (i, 0)),
        ],
        out_shape=[
            jax.ShapeDtypeStruct((E2, D), jnp.float32),
            jax.ShapeDtypeStruct((E2, D), jnp.float32),
        ],
    )(edge_attr, w, b)


def _up1(hu1, agg1, wu1t, b_up1, w2):
    return pl.pallas_call(
        _up1_body,
        grid=(NP // BN,),
        in_specs=[
            pl.BlockSpec((BN, D), lambda i: (i, 0)),
            pl.BlockSpec((2, BN, D), lambda i: (0, i, 0)),
            pl.BlockSpec((D, D), lambda i: (0, 0)),
            pl.BlockSpec((1, D), lambda i: (0, 0)),
            pl.BlockSpec((D, 2 * D), lambda i: (0, 0)),
        ],
        out_specs=[
            pl.BlockSpec((BN, D), lambda i: (i, 0)),
            pl.BlockSpec((BN, D), lambda i: (i, 0)),
        ],
        out_shape=[
            jax.ShapeDtypeStruct((NP, D), jnp.float32),
            jax.ShapeDtypeStruct((NP, D), jnp.float32),
        ],
    )(hu1, agg1, wu1t, b_up1, w2)


def _up2(hu2, agg2, wu2t, b_up2):
    return pl.pallas_call(
        _up2_body,
        grid=(NP // BN,),
        in_specs=[
            pl.BlockSpec((BN, D), lambda i: (i, 0)),
            pl.BlockSpec((2, BN, D), lambda i: (0, i, 0)),
            pl.BlockSpec((D, D), lambda i: (0, 0)),
            pl.BlockSpec((1, D), lambda i: (0, 0)),
        ],
        out_specs=pl.BlockSpec((1, D), lambda i: (0, 0)),
        out_shape=jax.ShapeDtypeStruct((1, D), jnp.float32),
    )(hu2, agg2, wu2t, b_up2)


def _sc_agg(hw, ew, srcr, dstr):
    mesh = plsc.VectorSubcoreMesh(core_axis_name="c", subcore_axis_name="s")

    @functools.partial(
        pl.kernel,
        mesh=mesh,
        out_type=jax.ShapeDtypeStruct((2, NP, D), jnp.float32),
        scratch_types=[
            pltpu.VMEM((CH,), jnp.int32),
            pltpu.VMEM((CH,), jnp.int32),
            pltpu.VMEM((CH,), jnp.int32),
            pltpu.VMEM((CH,), jnp.int32),
            pltpu.VMEM((CH,), jnp.int32),
            pltpu.VMEM((CH,), jnp.int32),
            pltpu.VMEM((CH, D), jnp.float32),
            pltpu.VMEM((CH, D), jnp.float32),
            pltpu.VMEM((CH, D), jnp.float32),
            pltpu.VMEM((CH, D), jnp.float32),
            pltpu.VMEM((CH, D), jnp.float32),
            pltpu.VMEM((CH, D), jnp.float32),
            pltpu.VMEM_SHARED((NA, D), jnp.float32),
            pltpu.SemaphoreType.DMA,
            pltpu.SemaphoreType.DMA,
            pltpu.SemaphoreType.DMA,
            pltpu.SemaphoreType.DMA,
            pltpu.SemaphoreType.DMA,
            pltpu.SemaphoreType.DMA,
            pltpu.SemaphoreType.DMA,
            pltpu.SemaphoreType.DMA,
            pltpu.SemaphoreType.DMA,
            pltpu.SemaphoreType.DMA,
            pltpu.SemaphoreType.DMA,
            pltpu.SemaphoreType.DMA,
        ],
    )
    def k(hw_hbm, ew_hbm, src_hbm, dst_hbm, out_hbm,
          si0, si1, si2, di0, di1, di2, gb0, gb1, gb2, mb0, mb1, mb2,
          agg_sh, g0, g1, g2, e0, e1, e2, s0, s1, s2, x0, x1, x2):
        c = lax.axis_index("c")
        s = lax.axis_index("s")
        w = c * 16 + s
        sis = (si0, si1, si2)
        dis = (di0, di1, di2)
        gbs = (gb0, gb1, gb2)
        mbs = (mb0, mb1, mb2)
        gsem = (g0, g1, g2)
        esem = (e0, e1, e2)
        ssem = (s0, s1, s2)
        xsem = (x0, x1, x2)

        def zrow(e_, carry):
            for j in range(D // 16):
                mb0[e_, pl.ds(16 * j, 16)] = jnp.zeros((16,), jnp.float32)
            return carry
        lax.fori_loop(0, CH, zrow, 0)
        for t in range(NWB64 // 16 + 1):
            wb = s * (NWB64 // 16 + 1) + t

            @pl.when(wb < NWB64)
            def _():
                pltpu.sync_copy(mb0, agg_sh.at[pl.ds(wb * CH, CH)])
        plsc.subcore_barrier()

        ebase = w * EPT2

        def idx_copies(i, b):
            a = pltpu.async_copy(src_hbm.at[pl.ds(ebase + i * CH, CH)],
                                 sis[b], xsem[b])
            bb = pltpu.async_copy(dst_hbm.at[pl.ds(ebase + i * CH, CH)],
                                  dis[b], xsem[b])
            return a, bb

        def wait_idx(i, b):
            a, bb = (pltpu.make_async_copy(
                         src_hbm.at[pl.ds(ebase + i * CH, CH)], sis[b],
                         xsem[b]),
                     pltpu.make_async_copy(
                         dst_hbm.at[pl.ds(ebase + i * CH, CH)], dis[b],
                         xsem[b]))
            a.wait()
            bb.wait()

        def issue_gather(i, b):
            pltpu.async_copy(hw_hbm.at[sis[b]], gbs[b], gsem[b])

        def issue_ew(i, b):
            pltpu.async_copy(ew_hbm.at[pl.ds(ebase + i * CH, CH)],
                             mbs[b], esem[b])

        def wait_scatter(b):
            pltpu.make_async_copy(mbs[b], agg_sh.at[dis[b]], ssem[b]).wait()

        idx_copies(0, 0)
        wait_idx(0, 0)
        issue_gather(0, 0)
        issue_ew(0, 0)
        idx_copies(1, 1)

        def sub(k_, j, b):
            i = 3 * k_ + j
            bn = (b + 1) % 3
            bp = (b + 2) % 3

            @pl.when(i + 1 < NCH)
            def _():
                wait_idx(i + 1, bn)
                issue_gather(i + 1, bn)
                issue_ew(i + 1, bn)

            @pl.when(i + 2 < NCH)
            def _():
                @pl.when(i >= 1)
                def _():
                    wait_scatter(bp)
                idx_copies(i + 2, bp)

            pltpu.make_async_copy(hw_hbm.at[sis[b]], gbs[b], gsem[b]).wait()
            pltpu.make_async_copy(ew_hbm.at[pl.ds(ebase + i * CH, CH)],
                                  mbs[b], esem[b]).wait()

            def crow(e_, carry):
                for jj in range(D // 16):
                    sl = pl.ds(16 * jj, 16)
                    mbs[b][e_, sl] = jnp.maximum(
                        mbs[b][e_, sl] + gbs[b][e_, sl], 0.0)
                return carry
            lax.fori_loop(0, CH, crow, 0, unroll=2)

            pltpu.async_copy(mbs[b], agg_sh.at[dis[b]], ssem[b], add=True)

        def body(k_, carry):
            sub(k_, 0, 0)
            sub(k_, 1, 1)
            sub(k_, 2, 2)
            return carry
        lax.fori_loop(0, NCH // 3, body, 0)

        wait_scatter(0)
        wait_scatter(1)
        wait_scatter(2)
        plsc.subcore_barrier()
        for t in range(NWB64 // 16 + 1):
            wb = s * (NWB64 // 16 + 1) + t

            @pl.when(wb < NWB64)
            def _():
                pltpu.sync_copy(agg_sh.at[pl.ds(wb * CH, CH)],
                                out_hbm.at[c, pl.ds(wb * CH, CH)])

    return k(hw, ew, srcr, dstr)


def kernel(x, edge_index, edge_attr, emb, W_msg1, b_msg1, W_msg2, b_msg2,
           W_up1, b_up1, W_up2, b_up2):
    srcr = jnp.pad(edge_index[0], (0, E2 - E))
    dstr = jnp.pad(edge_index[1], (0, E2 - E), constant_values=N)
    ea_pad = jnp.pad(edge_attr, ((0, E2 - E), (0, 0)))
    x_pad = jnp.pad(x, (0, NP - N)).reshape(NP, 1)

    t1 = emb @ W_msg1[:, :D].T
    tu1 = emb @ W_up1.T
    t = jnp.concatenate([t1, tu1], axis=1)
    we = jnp.concatenate([W_msg1[:, D:].T, W_msg2[:, D:].T], axis=1)
    be = jnp.concatenate([b_msg1, b_msg2]).reshape(1, 2 * D)
    w2 = jnp.concatenate([W_msg2[:, :D].T, W_up2.T], axis=1)

    hw1, hu1 = _prep(x_pad, t)
    ew1, ew2 = _edge_pre(ea_pad, we, be)

    agg1 = _sc_agg(hw1, ew1, srcr, dstr)
    hw2, hu2 = _up1(hu1, agg1, W_up1.T, b_up1.reshape(1, D), w2)
    agg2 = _sc_agg(hw2, ew2, srcr, dstr)
    out = _up2(hu2, agg2, W_up2.T, b_up2.reshape(1, D))
    return out[0] / N
